# baseline (speedup 1.0000x reference)
import jax
import jax.numpy as jnp
from jax import lax
from jax.experimental import pallas as pl

T = 32768
HIDDEN = 768
E = 8
K = 2
BT = 512
NSTREAM = 8
TQ = T // NSTREAM
NB = TQ // BT


def _router_block(w_ref, x_ref, lt_ref, st_ref, wt_ref, it_ref):
    lt = lax.dot_general(w_ref[...], x_ref[...],
                         (((1,), (1,)), ((), ())),
                         preferred_element_type=jnp.float32)
    lt_ref[...] = lt
    rows = [lt[e, :] for e in range(E)]
    m = rows[0]
    for e in range(1, E):
        m = jnp.maximum(m, rows[e])
    ex = [jnp.exp(r - m) for r in rows]
    s = ex[0]
    for e in range(1, E):
        s = s + ex[e]
    inv = 1.0 / s
    sc = [ev * inv for ev in ex]
    for e in range(E):
        st_ref[e, :] = sc[e]
    v1 = sc[0]
    for e in range(1, E):
        v1 = jnp.maximum(v1, sc[e])
    big = jnp.full((BT,), E, jnp.int32)
    i1 = big
    for e in range(E):
        i1 = jnp.minimum(i1, jnp.where(sc[e] == v1,
                                       jnp.full((BT,), e, jnp.int32), big))
    neg = jnp.full((BT,), -3.0e38, jnp.float32)
    v2 = neg
    for e in range(E):
        ecur = jnp.full((BT,), e, jnp.int32)
        v2 = jnp.maximum(v2, jnp.where(i1 == ecur, neg, sc[e]))
    i2 = big
    for e in range(E):
        ecur = jnp.full((BT,), e, jnp.int32)
        i2 = jnp.minimum(i2, jnp.where((sc[e] == v2) & (i1 != ecur),
                                       ecur, big))
    wt_ref[0, :] = v1
    wt_ref[1, :] = v2
    it_ref[0, :] = i1
    it_ref[1, :] = i2


def _body(*refs):
    w_ref = refs[0]
    x_refs = refs[1:1 + NSTREAM]
    out_refs = refs[1 + NSTREAM:]
    for j in range(NSTREAM):
        _router_block(w_ref, x_refs[j], out_refs[4 * j], out_refs[4 * j + 1],
                      out_refs[4 * j + 2], out_refs[4 * j + 3])


def _fused_router(w, x):
    out_specs = []
    out_shape = []
    for _ in range(NSTREAM):
        out_specs += [
            pl.BlockSpec((E, BT), lambda i: (0, i)),
            pl.BlockSpec((E, BT), lambda i: (0, i)),
            pl.BlockSpec((K, BT), lambda i: (0, i)),
            pl.BlockSpec((K, BT), lambda i: (0, i)),
        ]
        out_shape += [
            jax.ShapeDtypeStruct((E, TQ), jnp.float32),
            jax.ShapeDtypeStruct((E, TQ), jnp.float32),
            jax.ShapeDtypeStruct((K, TQ), jnp.float32),
            jax.ShapeDtypeStruct((K, TQ), jnp.int32),
        ]
    return pl.pallas_call(
        _body,
        grid=(NB,),
        in_specs=[pl.BlockSpec((E, HIDDEN), lambda i: (0, 0))] + [
            pl.BlockSpec((BT, HIDDEN), lambda i, j=j: (i + j * NB, 0))
            for j in range(NSTREAM)
        ],
        out_specs=out_specs,
        out_shape=out_shape,
    )(w, *([x] * NSTREAM))


@jax.jit
def kernel(x, W):
    outs = _fused_router(W, x)
    lts = [outs[4 * j] for j in range(NSTREAM)]
    sts = [outs[4 * j + 1] for j in range(NSTREAM)]
    wts = [outs[4 * j + 2] for j in range(NSTREAM)]
    its = [outs[4 * j + 3] for j in range(NSTREAM)]
    logits = jnp.concatenate(lts, axis=1).T
    scores = jnp.concatenate(sts, axis=1).T
    expert_weights = jnp.concatenate(wts, axis=1).T
    expert_indices = jnp.concatenate(its, axis=1).T
    return (scores, logits, expert_weights, expert_indices)


# speedup vs baseline: 1.5023x; 1.5023x over previous
import jax
import jax.numpy as jnp
from jax import lax
from jax.experimental import pallas as pl

T = 32768
HIDDEN = 768
E = 8
K = 2
BT = 1024
NSTREAM = 4
TQ = T // NSTREAM
NB = TQ // BT


def _router_block(w_ref, x_ref, lt_ref, st_ref, wt_ref, it_ref):
    lt = lax.dot_general(w_ref[...], x_ref[...],
                         (((1,), (1,)), ((), ())),
                         preferred_element_type=jnp.float32)
    lt_ref[...] = lt
    rows = [lt[e, :] for e in range(E)]
    m = rows[0]
    for e in range(1, E):
        m = jnp.maximum(m, rows[e])
    ex = [jnp.exp(r - m) for r in rows]
    s = ex[0]
    for e in range(1, E):
        s = s + ex[e]
    inv = 1.0 / s
    sc = [ev * inv for ev in ex]
    for e in range(E):
        st_ref[e, :] = sc[e]
    v1 = sc[0]
    for e in range(1, E):
        v1 = jnp.maximum(v1, sc[e])
    big = jnp.full((BT,), E, jnp.int32)
    i1 = big
    for e in range(E):
        i1 = jnp.minimum(i1, jnp.where(sc[e] == v1,
                                       jnp.full((BT,), e, jnp.int32), big))
    neg = jnp.full((BT,), -3.0e38, jnp.float32)
    v2 = neg
    for e in range(E):
        ecur = jnp.full((BT,), e, jnp.int32)
        v2 = jnp.maximum(v2, jnp.where(i1 == ecur, neg, sc[e]))
    i2 = big
    for e in range(E):
        ecur = jnp.full((BT,), e, jnp.int32)
        i2 = jnp.minimum(i2, jnp.where((sc[e] == v2) & (i1 != ecur),
                                       ecur, big))
    wt_ref[0, :] = v1
    wt_ref[1, :] = v2
    it_ref[0, :] = i1
    it_ref[1, :] = i2


def _body(*refs):
    w_ref = refs[0]
    x_refs = refs[1:1 + NSTREAM]
    out_refs = refs[1 + NSTREAM:]
    for j in range(NSTREAM):
        _router_block(w_ref, x_refs[j], out_refs[4 * j], out_refs[4 * j + 1],
                      out_refs[4 * j + 2], out_refs[4 * j + 3])


def _fused_router(w, x):
    out_specs = []
    out_shape = []
    for _ in range(NSTREAM):
        out_specs += [
            pl.BlockSpec((E, BT), lambda i: (0, i)),
            pl.BlockSpec((E, BT), lambda i: (0, i)),
            pl.BlockSpec((K, BT), lambda i: (0, i)),
            pl.BlockSpec((K, BT), lambda i: (0, i)),
        ]
        out_shape += [
            jax.ShapeDtypeStruct((E, TQ), jnp.float32),
            jax.ShapeDtypeStruct((E, TQ), jnp.float32),
            jax.ShapeDtypeStruct((K, TQ), jnp.float32),
            jax.ShapeDtypeStruct((K, TQ), jnp.int32),
        ]
    return pl.pallas_call(
        _body,
        grid=(NB,),
        in_specs=[pl.BlockSpec((E, HIDDEN), lambda i: (0, 0))] + [
            pl.BlockSpec((BT, HIDDEN), lambda i, j=j: (i + j * NB, 0))
            for j in range(NSTREAM)
        ],
        out_specs=out_specs,
        out_shape=out_shape,
    )(w, *([x] * NSTREAM))


@jax.jit
def kernel(x, W):
    outs = _fused_router(W, x)
    lts = [outs[4 * j] for j in range(NSTREAM)]
    sts = [outs[4 * j + 1] for j in range(NSTREAM)]
    wts = [outs[4 * j + 2] for j in range(NSTREAM)]
    its = [outs[4 * j + 3] for j in range(NSTREAM)]
    logits = jnp.concatenate(lts, axis=1).T
    scores = jnp.concatenate(sts, axis=1).T
    expert_weights = jnp.concatenate(wts, axis=1).T
    expert_indices = jnp.concatenate(its, axis=1).T
    return (scores, logits, expert_weights, expert_indices)
